# baseline (speedup 1.0000x reference)
import functools

import jax
import jax.numpy as jnp
from jax import lax
from jax.experimental import pallas as pl
from jax.experimental.pallas import tpu as pltpu
from jax.experimental.pallas import tpu_sc as plsc

_NC = 2
_NS = 16
_NW = _NC * _NS
_CH = 125
_GCH = 16


def _mesh():
    return plsc.VectorSubcoreMesh(core_axis_name="c", subcore_axis_name="s")


def _sc_degree(n, col3, zeros_n):
    nw, nch, ch = col3.shape
    ones_len = ((ch + 15) // 16) * 16

    def body(col_hbm, zero_hbm, out_hbm, col_v, ones_v, acc, sem):
        c = lax.axis_index("c")
        s = lax.axis_index("s")
        wid = s * _NC + c
        one16 = jnp.full((16,), 1.0, jnp.float32)
        for k in range(ones_len // 16):
            ones_v[pl.ds(16 * k, 16)] = one16

        @pl.when(s == 0)
        def _zero():
            pltpu.sync_copy(zero_hbm, acc)

        pltpu.async_copy(col_hbm.at[wid], col_v, sem).wait()
        plsc.subcore_barrier()

        def step(j, carry):
            pltpu.sync_copy(ones_v.at[pl.ds(0, ch)], acc.at[col_v.at[j]], add=True)
            return carry

        lax.fori_loop(0, nch, step, 0)
        plsc.subcore_barrier()

        @pl.when(s == 0)
        def _out():
            pltpu.sync_copy(acc, out_hbm.at[c])

    return pl.kernel(
        body,
        out_type=jax.ShapeDtypeStruct((_NC, n), jnp.float32),
        mesh=_mesh(),
        scratch_types=[
            pltpu.VMEM((nch, ch), jnp.int32),
            pltpu.VMEM((ones_len,), jnp.float32),
            pltpu.VMEM_SHARED((n,), jnp.float32),
            pltpu.SemaphoreType.DMA,
        ],
    )(col3, zeros_n)


def _sc_aggregate(n, d, zs, row3, col3, zeros_blk):
    nw, nch, ch = row3.shape
    base = (n // _NS) & ~7
    last = n - base * (_NS - 1)

    rps = n // _NS

    def body(zs_hbm, row_hbm, col_hbm, zero_hbm, out_hbm, row_v, col_v,
             gbuf0, gbuf1, acc, sem0, sem1):
        c = lax.axis_index("c")
        s = lax.axis_index("s")
        wid = s * _NC + c

        pltpu.async_copy(zero_hbm, gbuf0, sem0).wait()
        for i in range(rps // ch + (1 if rps % ch else 0)):
            span = min(ch, rps - i * ch)
            pltpu.sync_copy(
                gbuf0.at[pl.ds(0, span)],
                acc.at[pl.ds(s * rps + i * ch, span)],
            )

        plsc.subcore_barrier()

        def group(g, carry):
            pltpu.async_copy(row_hbm.at[wid, pl.ds(g * _GCH, _GCH)], row_v, sem0).wait()
            pltpu.async_copy(col_hbm.at[wid, pl.ds(g * _GCH, _GCH)], col_v, sem0).wait()
            pltpu.async_copy(zs_hbm.at[row_v.at[0]], gbuf0, sem0)

            def step(t, c2):
                j0 = 2 * t
                cp1 = pltpu.async_copy(zs_hbm.at[row_v.at[j0 + 1]], gbuf1, sem1)
                pltpu.make_async_copy(zs_hbm.at[row_v.at[j0]], gbuf0, sem0).wait()
                pltpu.sync_copy(gbuf0, acc.at[col_v.at[j0]], add=True)

                @pl.when(t < _GCH // 2 - 1)
                def _next():
                    pltpu.async_copy(zs_hbm.at[row_v.at[j0 + 2]], gbuf0, sem0)

                cp1.wait()
                pltpu.sync_copy(gbuf1, acc.at[col_v.at[j0 + 1]], add=True)
                return c2

            lax.fori_loop(0, _GCH // 2, step, 0)
            return carry

        lax.fori_loop(0, nch // _GCH, group, 0)
        plsc.subcore_barrier()

        @pl.when(s < _NS - 1)
        def _out_main():
            pltpu.sync_copy(acc.at[pl.ds(s * base, base)], out_hbm.at[c, pl.ds(s * base, base)])

        @pl.when(s == _NS - 1)
        def _out_last():
            pltpu.sync_copy(
                acc.at[pl.ds(base * (_NS - 1), last)],
                out_hbm.at[c, pl.ds(base * (_NS - 1), last)],
            )

    return pl.kernel(
        body,
        out_type=jax.ShapeDtypeStruct((_NC, n, d), jnp.float32),
        mesh=_mesh(),
        scratch_types=[
            pltpu.VMEM((_GCH, ch), jnp.int32),
            pltpu.VMEM((_GCH, ch), jnp.int32),
            pltpu.VMEM((ch, d), jnp.float32),
            pltpu.VMEM((ch, d), jnp.float32),
            pltpu.VMEM_SHARED((n, d), jnp.float32),
            pltpu.SemaphoreType.DMA,
            pltpu.SemaphoreType.DMA,
        ],
    )(zs, row3, col3, zeros_blk)


def _tc(fn, out_shape, *args):
    return pl.pallas_call(fn, out_shape=out_shape)(*args)


def _tc_stage_in(degT, x, W1, b1r):
    def body(deg_ref, x_ref, w1_ref, b1_ref, zs1_ref, dis_ref):
        deg = deg_ref[:, 0:1] + deg_ref[:, 1:2] + 1.0
        dis = lax.rsqrt(deg)
        dis_ref[...] = dis
        zs1_ref[...] = dis * (
            jnp.dot(x_ref[...], w1_ref[...], preferred_element_type=jnp.float32)
            + b1_ref[...]
        )

    n = x.shape[0]
    dh = W1.shape[1]
    return _tc(
        body,
        (
            jax.ShapeDtypeStruct((n, dh), jnp.float32),
            jax.ShapeDtypeStruct((n, 1), jnp.float32),
        ),
        degT, x, W1, b1r,
    )


def _tc_stage_mid(aggp, zs_prev, dis, W, br):
    def body(agg_ref, zs_ref, dis_ref, w_ref, b_ref, out_ref):
        dis = dis_ref[...]
        h = jnp.maximum(dis * (agg_ref[0] + agg_ref[1] + zs_ref[...]), 0.0)
        out_ref[...] = dis * (
            jnp.dot(h, w_ref[...], preferred_element_type=jnp.float32) + b_ref[...]
        )

    n = zs_prev.shape[0]
    dout = W.shape[1]
    return _tc(
        body,
        jax.ShapeDtypeStruct((n, dout), jnp.float32),
        aggp, zs_prev, dis, W, br,
    )


def _tc_stage_out(aggp, zs3, dis, Wo, bor):
    def body(agg_ref, zs_ref, dis_ref, wo_ref, b_ref, out_ref):
        h = dis_ref[...] * (agg_ref[0] + agg_ref[1] + zs_ref[...])
        out_ref[...] = (
            jnp.dot(h, wo_ref[...], preferred_element_type=jnp.float32) + b_ref[...]
        )

    n = zs3.shape[0]
    df = Wo.shape[1]
    return _tc(
        body,
        jax.ShapeDtypeStruct((n, df), jnp.float32),
        aggp, zs3, dis, Wo, bor,
    )


def kernel(x, edge_index, W1, b1, W2, b2, W3, b3, Wo, bo):
    n, din = x.shape
    e = edge_index.shape[1]
    ew = e // _NW
    nch = ew // _CH
    assert ew * _NW == e and nch * _CH == ew

    row3 = edge_index[0].reshape(_NW, nch, _CH)
    col3 = edge_index[1].reshape(_NW, nch, _CH)
    zeros_n = jnp.zeros((n,), jnp.float32)

    zeros_blk = jnp.zeros((_CH, W1.shape[1]), jnp.float32)
    b1r = b1.reshape(1, -1)
    b2r = b2.reshape(1, -1)
    b3r = b3.reshape(1, -1)
    bor = bo.reshape(1, -1)

    deg2 = _sc_degree(n, col3, zeros_n)
    zs1, dis = _tc_stage_in(deg2.T, x, W1, b1r)
    agg1 = _sc_aggregate(n, W1.shape[1], zs1, row3, col3, zeros_blk)
    zs2 = _tc_stage_mid(agg1, zs1, dis, W2, b2r)
    agg2 = _sc_aggregate(n, W2.shape[1], zs2, row3, col3, zeros_blk)
    zs3 = _tc_stage_mid(agg2, zs2, dis, W3, b3r)
    agg3 = _sc_aggregate(n, W3.shape[1], zs3, row3, col3, zeros_blk)
    return _tc_stage_out(agg3, zs3, dis, Wo, bor)


# speedup vs baseline: 1.0768x; 1.0768x over previous
import functools

import jax
import jax.numpy as jnp
from jax import lax
from jax.experimental import pallas as pl
from jax.experimental.pallas import tpu as pltpu
from jax.experimental.pallas import tpu_sc as plsc

_NC = 2
_NS = 16
_NW = _NC * _NS
_CH = 125
_GCH = 16


def _mesh():
    return plsc.VectorSubcoreMesh(core_axis_name="c", subcore_axis_name="s")


def _sc_degree(n, col3, zeros_n):
    nw, nch, ch = col3.shape
    ones_len = ((ch + 15) // 16) * 16

    def body(col_hbm, zero_hbm, out_hbm, col_v, ones_v, acc, sem):
        c = lax.axis_index("c")
        s = lax.axis_index("s")
        wid = s * _NC + c
        one16 = jnp.full((16,), 1.0, jnp.float32)
        for k in range(ones_len // 16):
            ones_v[pl.ds(16 * k, 16)] = one16

        @pl.when(s == 0)
        def _zero():
            pltpu.sync_copy(zero_hbm, acc)

        pltpu.async_copy(col_hbm.at[wid], col_v, sem).wait()
        plsc.subcore_barrier()

        def step(j, carry):
            pltpu.sync_copy(ones_v.at[pl.ds(0, ch)], acc.at[col_v.at[j]], add=True)
            return carry

        lax.fori_loop(0, nch, step, 0)
        plsc.subcore_barrier()

        @pl.when(s == 0)
        def _out():
            pltpu.sync_copy(acc, out_hbm.at[c])

    return pl.kernel(
        body,
        out_type=jax.ShapeDtypeStruct((_NC, n), jnp.float32),
        mesh=_mesh(),
        scratch_types=[
            pltpu.VMEM((nch, ch), jnp.int32),
            pltpu.VMEM((ones_len,), jnp.float32),
            pltpu.VMEM_SHARED((n,), jnp.float32),
            pltpu.SemaphoreType.DMA,
        ],
    )(col3, zeros_n)


def _sc_aggregate(n, d, zs, row3, col3, zeros_blk):
    nw, nch, ch = row3.shape
    base = (n // _NS) & ~7
    last = n - base * (_NS - 1)

    rps = n // _NS

    def body(zs_hbm, row_hbm, col_hbm, zero_hbm, out_hbm, row_v, col_v,
             gbuf0, gbuf1, acc, sem0, sem1):
        c = lax.axis_index("c")
        s = lax.axis_index("s")
        wid = s * _NC + c

        pltpu.async_copy(zero_hbm, gbuf0, sem0).wait()
        for i in range(rps // ch + (1 if rps % ch else 0)):
            span = min(ch, rps - i * ch)
            pltpu.sync_copy(
                gbuf0.at[pl.ds(0, span)],
                acc.at[pl.ds(s * rps + i * ch, span)],
            )

        plsc.subcore_barrier()

        def group(g, carry):
            pltpu.async_copy(row_hbm.at[wid, pl.ds(g * _GCH, _GCH)], row_v, sem0).wait()
            pltpu.async_copy(col_hbm.at[wid, pl.ds(g * _GCH, _GCH)], col_v, sem0).wait()
            pltpu.async_copy(zs_hbm.at[row_v.at[0]], gbuf0, sem0)

            def step(t, c2):
                j0 = 2 * t
                cp1 = pltpu.async_copy(zs_hbm.at[row_v.at[j0 + 1]], gbuf1, sem1)
                pltpu.make_async_copy(zs_hbm.at[row_v.at[j0]], gbuf0, sem0).wait()
                pltpu.sync_copy(gbuf0, acc.at[col_v.at[j0]], add=True)

                @pl.when(t < _GCH // 2 - 1)
                def _next():
                    pltpu.async_copy(zs_hbm.at[row_v.at[j0 + 2]], gbuf0, sem0)

                cp1.wait()
                pltpu.sync_copy(gbuf1, acc.at[col_v.at[j0 + 1]], add=True)
                return c2

            lax.fori_loop(0, _GCH // 2, step, 0)
            return carry

        lax.fori_loop(0, nch // _GCH, group, 0)
        plsc.subcore_barrier()

        @pl.when(s < _NS - 1)
        def _out_main():
            pltpu.sync_copy(acc.at[pl.ds(s * base, base)], out_hbm.at[c, pl.ds(s * base, base)])

        @pl.when(s == _NS - 1)
        def _out_last():
            pltpu.sync_copy(
                acc.at[pl.ds(base * (_NS - 1), last)],
                out_hbm.at[c, pl.ds(base * (_NS - 1), last)],
            )

    dt = zs.dtype
    return pl.kernel(
        body,
        out_type=jax.ShapeDtypeStruct((_NC, n, d), dt),
        mesh=_mesh(),
        compiler_params=pltpu.CompilerParams(use_tc_tiling_on_sc=False),
        scratch_types=[
            pltpu.VMEM((_GCH, ch), jnp.int32),
            pltpu.VMEM((_GCH, ch), jnp.int32),
            pltpu.VMEM((ch, d), dt),
            pltpu.VMEM((ch, d), dt),
            pltpu.VMEM_SHARED((n, d), dt),
            pltpu.SemaphoreType.DMA,
            pltpu.SemaphoreType.DMA,
        ],
    )(zs, row3, col3, zeros_blk)


def _tc(fn, out_shape, *args):
    return pl.pallas_call(fn, out_shape=out_shape)(*args)


def _tc_stage_in(degT, x, W1, b1r):
    def body(deg_ref, x_ref, w1_ref, b1_ref, zs1_ref, dis_ref):
        deg = deg_ref[:, 0:1] + deg_ref[:, 1:2] + 1.0
        dis = lax.rsqrt(deg)
        dis_ref[...] = dis
        zs1_ref[...] = (dis * (
            jnp.dot(x_ref[...], w1_ref[...], preferred_element_type=jnp.float32)
            + b1_ref[...]
        )).astype(zs1_ref.dtype)

    n = x.shape[0]
    dh = W1.shape[1]
    return _tc(
        body,
        (
            jax.ShapeDtypeStruct((n, dh), jnp.bfloat16),
            jax.ShapeDtypeStruct((n, 1), jnp.float32),
        ),
        degT, x, W1, b1r,
    )


def _tc_stage_mid(aggp, zs_prev, dis, W, br):
    def body(agg_ref, zs_ref, dis_ref, w_ref, b_ref, out_ref):
        dis = dis_ref[...]
        tot = (agg_ref[0].astype(jnp.float32) + agg_ref[1].astype(jnp.float32)
               + zs_ref[...].astype(jnp.float32))
        h = jnp.maximum(dis * tot, 0.0)
        out_ref[...] = (dis * (
            jnp.dot(h, w_ref[...], preferred_element_type=jnp.float32) + b_ref[...]
        )).astype(out_ref.dtype)

    n = zs_prev.shape[0]
    dout = W.shape[1]
    return _tc(
        body,
        jax.ShapeDtypeStruct((n, dout), jnp.bfloat16),
        aggp, zs_prev, dis, W, br,
    )


def _tc_stage_out(aggp, zs3, dis, Wo, bor):
    def body(agg_ref, zs_ref, dis_ref, wo_ref, b_ref, out_ref):
        h = dis_ref[...] * (agg_ref[0].astype(jnp.float32)
                            + agg_ref[1].astype(jnp.float32)
                            + zs_ref[...].astype(jnp.float32))
        out_ref[...] = (
            jnp.dot(h, wo_ref[...], preferred_element_type=jnp.float32) + b_ref[...]
        )

    n = zs3.shape[0]
    df = Wo.shape[1]
    return _tc(
        body,
        jax.ShapeDtypeStruct((n, df), jnp.float32),
        aggp, zs3, dis, Wo, bor,
    )


def kernel(x, edge_index, W1, b1, W2, b2, W3, b3, Wo, bo):
    n, din = x.shape
    e = edge_index.shape[1]
    ew = e // _NW
    nch = ew // _CH
    assert ew * _NW == e and nch * _CH == ew

    row3 = edge_index[0].reshape(_NW, nch, _CH)
    col3 = edge_index[1].reshape(_NW, nch, _CH)
    zeros_n = jnp.zeros((n,), jnp.float32)

    zeros_blk = jnp.zeros((_CH, W1.shape[1]), jnp.bfloat16)
    b1r = b1.reshape(1, -1)
    b2r = b2.reshape(1, -1)
    b3r = b3.reshape(1, -1)
    bor = bo.reshape(1, -1)

    deg2 = _sc_degree(n, col3, zeros_n)
    zs1, dis = _tc_stage_in(deg2.T, x, W1, b1r)
    agg1 = _sc_aggregate(n, W1.shape[1], zs1, row3, col3, zeros_blk)
    zs2 = _tc_stage_mid(agg1, zs1, dis, W2, b2r)
    agg2 = _sc_aggregate(n, W2.shape[1], zs2, row3, col3, zeros_blk)
    zs3 = _tc_stage_mid(agg2, zs2, dis, W3, b3r)
    agg3 = _sc_aggregate(n, W3.shape[1], zs3, row3, col3, zeros_blk)
    return _tc_stage_out(agg3, zs3, dis, Wo, bor)


# speedup vs baseline: 1.2682x; 1.1778x over previous
import functools

import jax
import jax.numpy as jnp
from jax import lax
from jax.experimental import pallas as pl
from jax.experimental.pallas import tpu as pltpu
from jax.experimental.pallas import tpu_sc as plsc

_NC = 2
_NS = 16
_NW = _NC * _NS
_CH = 125
_GCH = 16


def _mesh():
    return plsc.VectorSubcoreMesh(core_axis_name="c", subcore_axis_name="s")


def _sc_degree(n, col3, zeros_n):
    nw, nch, ch = col3.shape
    ones_len = ((ch + 15) // 16) * 16

    def body(col_hbm, zero_hbm, out_hbm, col_v, ones_v, acc, sem):
        c = lax.axis_index("c")
        s = lax.axis_index("s")
        wid = s * _NC + c
        one16 = jnp.full((16,), 1.0, jnp.float32)
        for k in range(ones_len // 16):
            ones_v[pl.ds(16 * k, 16)] = one16

        @pl.when(s == 0)
        def _zero():
            pltpu.sync_copy(zero_hbm, acc)

        pltpu.async_copy(col_hbm.at[wid], col_v, sem).wait()
        plsc.subcore_barrier()

        def step(j, carry):
            pltpu.sync_copy(ones_v.at[pl.ds(0, ch)], acc.at[col_v.at[j]], add=True)
            return carry

        lax.fori_loop(0, nch, step, 0)
        plsc.subcore_barrier()

        @pl.when(s == 0)
        def _out():
            pltpu.sync_copy(acc, out_hbm.at[c])

    return pl.kernel(
        body,
        out_type=jax.ShapeDtypeStruct((_NC, n), jnp.float32),
        mesh=_mesh(),
        scratch_types=[
            pltpu.VMEM((nch, ch), jnp.int32),
            pltpu.VMEM((ones_len,), jnp.float32),
            pltpu.VMEM_SHARED((n,), jnp.float32),
            pltpu.SemaphoreType.DMA,
        ],
    )(col3, zeros_n)


def _sc_aggregate(n, d, zs, row3, col3, zeros_blk):
    nw, nch, ch = row3.shape
    base = (n // _NS) & ~7
    last = n - base * (_NS - 1)

    rps = n // _NS

    def body(zs_hbm, row_hbm, col_hbm, zero_hbm, out_hbm, row_v, col_v,
             gb0, gb1, gb2, gb3, acc, sm0, sm1, sm2, sm3):
        c = lax.axis_index("c")
        s = lax.axis_index("s")
        wid = s * _NC + c
        gbufs = (gb0, gb1, gb2, gb3)
        sems = (sm0, sm1, sm2, sm3)

        pltpu.async_copy(zero_hbm, gb0, sm0).wait()
        for i in range(rps // ch + (1 if rps % ch else 0)):
            span = min(ch, rps - i * ch)
            pltpu.sync_copy(
                gb0.at[pl.ds(0, span)],
                acc.at[pl.ds(s * rps + i * ch, span)],
            )

        pltpu.async_copy(row_hbm.at[wid], row_v, sm1).wait()
        pltpu.async_copy(col_hbm.at[wid], col_v, sm1).wait()
        plsc.subcore_barrier()

        for k in range(3):
            pltpu.async_copy(zs_hbm.at[row_v.at[k]], gbufs[k], sems[k])

        def step(t, c2):
            j0 = 4 * t
            for k in range(4):
                pltpu.make_async_copy(zs_hbm.at[row_v.at[j0 + k]], gbufs[k], sems[k]).wait()
                pltpu.sync_copy(gbufs[k], acc.at[col_v.at[j0 + k]], add=True)
                nxt = j0 + k + 3
                kn = (k + 3) % 4

                @pl.when(nxt < nch)
                def _issue(nxt=nxt, kn=kn):
                    pltpu.async_copy(zs_hbm.at[row_v.at[nxt]], gbufs[kn], sems[kn])

            return c2

        lax.fori_loop(0, nch // 4, step, 0)
        plsc.subcore_barrier()

        @pl.when(s < _NS - 1)
        def _out_main():
            pltpu.sync_copy(acc.at[pl.ds(s * base, base)], out_hbm.at[c, pl.ds(s * base, base)])

        @pl.when(s == _NS - 1)
        def _out_last():
            pltpu.sync_copy(
                acc.at[pl.ds(base * (_NS - 1), last)],
                out_hbm.at[c, pl.ds(base * (_NS - 1), last)],
            )

    dt = zs.dtype
    return pl.kernel(
        body,
        out_type=jax.ShapeDtypeStruct((_NC, n, d), dt),
        mesh=_mesh(),
        compiler_params=pltpu.CompilerParams(use_tc_tiling_on_sc=False),
        scratch_types=[
            pltpu.VMEM((nch, ch), jnp.int32),
            pltpu.VMEM((nch, ch), jnp.int32),
            pltpu.VMEM((ch, d), dt),
            pltpu.VMEM((ch, d), dt),
            pltpu.VMEM((ch, d), dt),
            pltpu.VMEM((ch, d), dt),
            pltpu.VMEM_SHARED((n, d), dt),
            pltpu.SemaphoreType.DMA,
            pltpu.SemaphoreType.DMA,
            pltpu.SemaphoreType.DMA,
            pltpu.SemaphoreType.DMA,
        ],
    )(zs, row3, col3, zeros_blk)


def _tc(fn, out_shape, *args):
    return pl.pallas_call(fn, out_shape=out_shape)(*args)


def _tc_stage_in(degT, x, W1, b1r):
    def body(deg_ref, x_ref, w1_ref, b1_ref, zs1_ref, dis_ref):
        deg = deg_ref[:, 0:1] + deg_ref[:, 1:2] + 1.0
        dis = lax.rsqrt(deg)
        dis_ref[...] = dis
        zs1_ref[...] = (dis * (
            jnp.dot(x_ref[...], w1_ref[...], preferred_element_type=jnp.float32)
            + b1_ref[...]
        )).astype(zs1_ref.dtype)

    n = x.shape[0]
    dh = W1.shape[1]
    return _tc(
        body,
        (
            jax.ShapeDtypeStruct((n, dh), jnp.bfloat16),
            jax.ShapeDtypeStruct((n, 1), jnp.float32),
        ),
        degT, x, W1, b1r,
    )


def _tc_stage_mid(aggp, zs_prev, dis, W, br):
    def body(agg_ref, zs_ref, dis_ref, w_ref, b_ref, out_ref):
        dis = dis_ref[...]
        tot = (agg_ref[0].astype(jnp.float32) + agg_ref[1].astype(jnp.float32)
               + zs_ref[...].astype(jnp.float32))
        h = jnp.maximum(dis * tot, 0.0)
        out_ref[...] = (dis * (
            jnp.dot(h, w_ref[...], preferred_element_type=jnp.float32) + b_ref[...]
        )).astype(out_ref.dtype)

    n = zs_prev.shape[0]
    dout = W.shape[1]
    return _tc(
        body,
        jax.ShapeDtypeStruct((n, dout), jnp.bfloat16),
        aggp, zs_prev, dis, W, br,
    )


def _tc_stage_out(aggp, zs3, dis, Wo, bor):
    def body(agg_ref, zs_ref, dis_ref, wo_ref, b_ref, out_ref):
        h = dis_ref[...] * (agg_ref[0].astype(jnp.float32)
                            + agg_ref[1].astype(jnp.float32)
                            + zs_ref[...].astype(jnp.float32))
        out_ref[...] = (
            jnp.dot(h, wo_ref[...], preferred_element_type=jnp.float32) + b_ref[...]
        )

    n = zs3.shape[0]
    df = Wo.shape[1]
    return _tc(
        body,
        jax.ShapeDtypeStruct((n, df), jnp.float32),
        aggp, zs3, dis, Wo, bor,
    )


def kernel(x, edge_index, W1, b1, W2, b2, W3, b3, Wo, bo):
    n, din = x.shape
    e = edge_index.shape[1]
    ew = e // _NW
    nch = ew // _CH
    assert ew * _NW == e and nch * _CH == ew

    row3 = edge_index[0].reshape(_NW, nch, _CH)
    col3 = edge_index[1].reshape(_NW, nch, _CH)
    zeros_n = jnp.zeros((n,), jnp.float32)

    zeros_blk = jnp.zeros((_CH, W1.shape[1]), jnp.bfloat16)
    b1r = b1.reshape(1, -1)
    b2r = b2.reshape(1, -1)
    b3r = b3.reshape(1, -1)
    bor = bo.reshape(1, -1)

    deg2 = _sc_degree(n, col3, zeros_n)
    zs1, dis = _tc_stage_in(deg2.T, x, W1, b1r)
    agg1 = _sc_aggregate(n, W1.shape[1], zs1, row3, col3, zeros_blk)
    zs2 = _tc_stage_mid(agg1, zs1, dis, W2, b2r)
    agg2 = _sc_aggregate(n, W2.shape[1], zs2, row3, col3, zeros_blk)
    zs3 = _tc_stage_mid(agg2, zs2, dis, W3, b3r)
    agg3 = _sc_aggregate(n, W3.shape[1], zs3, row3, col3, zeros_blk)
    return _tc_stage_out(agg3, zs3, dis, Wo, bor)


# speedup vs baseline: 1.2688x; 1.0004x over previous
import functools

import jax
import jax.numpy as jnp
from jax import lax
from jax.experimental import pallas as pl
from jax.experimental.pallas import tpu as pltpu
from jax.experimental.pallas import tpu_sc as plsc

_NC = 2
_NS = 16
_NW = _NC * _NS
_CH = 125


def _mesh():
    return plsc.VectorSubcoreMesh(core_axis_name="c", subcore_axis_name="s")


def _sc_degree(n, col3, zeros_n):
    nw, nch, ch = col3.shape
    ones_len = ((ch + 15) // 16) * 16

    def body(col_hbm, zero_hbm, out_hbm, col_v, ones_v, acc, sem):
        c = lax.axis_index("c")
        s = lax.axis_index("s")
        wid = s * _NC + c
        one16 = jnp.full((16,), 1.0, jnp.float32)
        for k in range(ones_len // 16):
            ones_v[pl.ds(16 * k, 16)] = one16

        @pl.when(s == 0)
        def _zero():
            pltpu.sync_copy(zero_hbm, acc)

        pltpu.async_copy(col_hbm.at[wid], col_v, sem).wait()
        plsc.subcore_barrier()

        def step(j, carry):
            pltpu.sync_copy(ones_v.at[pl.ds(0, ch)], acc.at[col_v.at[j]], add=True)
            return carry

        lax.fori_loop(0, nch, step, 0)
        plsc.subcore_barrier()

        @pl.when(s == 0)
        def _out():
            pltpu.sync_copy(acc, out_hbm.at[c])

    return pl.kernel(
        body,
        out_type=jax.ShapeDtypeStruct((_NC, n), jnp.float32),
        mesh=_mesh(),
        scratch_types=[
            pltpu.VMEM((nch, ch), jnp.int32),
            pltpu.VMEM((ones_len,), jnp.float32),
            pltpu.VMEM_SHARED((n,), jnp.float32),
            pltpu.SemaphoreType.DMA,
        ],
    )(col3, zeros_n)


def _sc_aggregate(n, d, zs, row3, col3, zeros_blk):
    nw, nch, ch = row3.shape
    base = (n // _NS) & ~7
    last = n - base * (_NS - 1)

    rps = n // _NS

    def body(zs_hbm, row_hbm, col_hbm, zero_hbm, out_hbm, row_v, col_v,
             gb0, gb1, gb2, gb3, acc, sm0, sm1, sm2, sm3):
        c = lax.axis_index("c")
        s = lax.axis_index("s")
        wid = s * _NC + c
        gbufs = (gb0, gb1, gb2, gb3)
        sems = (sm0, sm1, sm2, sm3)

        pltpu.async_copy(zero_hbm, gb0, sm0).wait()
        for i in range(rps // ch + (1 if rps % ch else 0)):
            span = min(ch, rps - i * ch)
            pltpu.sync_copy(
                gb0.at[pl.ds(0, span)],
                acc.at[pl.ds(s * rps + i * ch, span)],
            )

        pltpu.async_copy(row_hbm.at[wid], row_v, sm1).wait()
        pltpu.async_copy(col_hbm.at[wid], col_v, sm1).wait()
        plsc.subcore_barrier()

        for k in range(3):
            pltpu.async_copy(zs_hbm.at[row_v.at[k]], gbufs[k], sems[k])

        def step(t, c2):
            j0 = 4 * t
            for k in range(4):
                pltpu.make_async_copy(zs_hbm.at[row_v.at[j0 + k]], gbufs[k], sems[k]).wait()
                pltpu.sync_copy(gbufs[k], acc.at[col_v.at[j0 + k]], add=True)
                nxt = j0 + k + 3
                kn = (k + 3) % 4

                @pl.when(nxt < nch)
                def _issue(nxt=nxt, kn=kn):
                    pltpu.async_copy(zs_hbm.at[row_v.at[nxt]], gbufs[kn], sems[kn])

            return c2

        lax.fori_loop(0, nch // 4, step, 0)
        plsc.subcore_barrier()

        @pl.when(s < _NS - 1)
        def _out_main():
            pltpu.sync_copy(acc.at[pl.ds(s * base, base)], out_hbm.at[c, pl.ds(s * base, base)])

        @pl.when(s == _NS - 1)
        def _out_last():
            pltpu.sync_copy(
                acc.at[pl.ds(base * (_NS - 1), last)],
                out_hbm.at[c, pl.ds(base * (_NS - 1), last)],
            )

    dt = zs.dtype
    return pl.kernel(
        body,
        out_type=jax.ShapeDtypeStruct((_NC, n, d), dt),
        mesh=_mesh(),
        compiler_params=pltpu.CompilerParams(use_tc_tiling_on_sc=False),
        scratch_types=[
            pltpu.VMEM((nch, ch), jnp.int32),
            pltpu.VMEM((nch, ch), jnp.int32),
            pltpu.VMEM((ch, d), dt),
            pltpu.VMEM((ch, d), dt),
            pltpu.VMEM((ch, d), dt),
            pltpu.VMEM((ch, d), dt),
            pltpu.VMEM_SHARED((n, d), dt),
            pltpu.SemaphoreType.DMA,
            pltpu.SemaphoreType.DMA,
            pltpu.SemaphoreType.DMA,
            pltpu.SemaphoreType.DMA,
        ],
    )(zs, row3, col3, zeros_blk)


def _tc(fn, out_shape, *args):
    return pl.pallas_call(fn, out_shape=out_shape)(*args)


def _tc_stage_in(degT, x, W1, b1r):
    def body(deg_ref, x_ref, w1_ref, b1_ref, zs1_ref, dis_ref):
        deg = deg_ref[:, 0:1] + deg_ref[:, 1:2] + 1.0
        dis = lax.rsqrt(deg)
        dis_ref[...] = dis
        zs1_ref[...] = (dis * (
            jnp.dot(x_ref[...], w1_ref[...], preferred_element_type=jnp.float32)
            + b1_ref[...]
        )).astype(zs1_ref.dtype)

    n = x.shape[0]
    dh = W1.shape[1]
    return _tc(
        body,
        (
            jax.ShapeDtypeStruct((n, dh), jnp.bfloat16),
            jax.ShapeDtypeStruct((n, 1), jnp.float32),
        ),
        degT, x, W1, b1r,
    )


def _tc_stage_mid(aggp, zs_prev, dis, W, br):
    def body(agg_ref, zs_ref, dis_ref, w_ref, b_ref, out_ref):
        dis = dis_ref[...]
        tot = (agg_ref[0].astype(jnp.float32) + agg_ref[1].astype(jnp.float32)
               + zs_ref[...].astype(jnp.float32))
        h = jnp.maximum(dis * tot, 0.0)
        out_ref[...] = (dis * (
            jnp.dot(h, w_ref[...], preferred_element_type=jnp.float32) + b_ref[...]
        )).astype(out_ref.dtype)

    n = zs_prev.shape[0]
    dout = W.shape[1]
    return _tc(
        body,
        jax.ShapeDtypeStruct((n, dout), jnp.bfloat16),
        aggp, zs_prev, dis, W, br,
    )


def _tc_stage_out(aggp, zs3, dis, Wo, bor):
    def body(agg_ref, zs_ref, dis_ref, wo_ref, b_ref, out_ref):
        h = dis_ref[...] * (agg_ref[0].astype(jnp.float32)
                            + agg_ref[1].astype(jnp.float32)
                            + zs_ref[...].astype(jnp.float32))
        out_ref[...] = (
            jnp.dot(h, wo_ref[...], preferred_element_type=jnp.float32) + b_ref[...]
        )

    n = zs3.shape[0]
    df = Wo.shape[1]
    return _tc(
        body,
        jax.ShapeDtypeStruct((n, df), jnp.float32),
        aggp, zs3, dis, Wo, bor,
    )


def kernel(x, edge_index, W1, b1, W2, b2, W3, b3, Wo, bo):
    n, din = x.shape
    e = edge_index.shape[1]
    ew = e // _NW
    nch = ew // _CH
    assert ew * _NW == e and nch * _CH == ew

    row3 = edge_index[0].reshape(_NW, nch, _CH)
    col3 = edge_index[1].reshape(_NW, nch, _CH)
    zeros_n = jnp.zeros((n,), jnp.float32)

    zeros_blk = jnp.zeros((_CH, W1.shape[1]), jnp.bfloat16)
    b1r = b1.reshape(1, -1)
    b2r = b2.reshape(1, -1)
    b3r = b3.reshape(1, -1)
    bor = bo.reshape(1, -1)

    deg2 = _sc_degree(n, col3, zeros_n)
    zs1, dis = _tc_stage_in(deg2.T, x, W1, b1r)
    agg1 = _sc_aggregate(n, W1.shape[1], zs1, row3, col3, zeros_blk)
    zs2 = _tc_stage_mid(agg1, zs1, dis, W2, b2r)
    agg2 = _sc_aggregate(n, W2.shape[1], zs2, row3, col3, zeros_blk)
    zs3 = _tc_stage_mid(agg2, zs2, dis, W3, b3r)
    agg3 = _sc_aggregate(n, W3.shape[1], zs3, row3, col3, zeros_blk)
    return _tc_stage_out(agg3, zs3, dis, Wo, bor)
